# baseline (speedup 1.0000x reference)
import functools

import jax
import jax.numpy as jnp
from jax import lax
from jax.experimental import pallas as pl
from jax.experimental.pallas import tpu as pltpu
from jax.experimental.pallas import tpu_sc as plsc

NUM_ROWS = 50000
NUM_COLS = 256
NUM_SEGS = 64
LANES = 16
VREGS_PER_ROW = NUM_COLS // LANES
NUM_CORES = 2
NUM_SUBCORES = 16
NUM_WORKERS = NUM_CORES * NUM_SUBCORES
SEGS_PER_WORKER = NUM_SEGS // NUM_WORKERS
CHUNK_ROWS = 80
NBUF = 4


def _sc_body(x_hbm, starts_hbm, out_hbm, starts_v, buf, out_v, sems):
    w = lax.axis_index("s") * NUM_CORES + lax.axis_index("c")
    pltpu.sync_copy(starts_hbm, starts_v)
    bounds = starts_v[pl.ds(SEGS_PER_WORKER * w, LANES)]

    lo = bounds[0]
    mid = bounds[1]
    hi = bounds[2]
    a = (lo // 8) * 8
    nch = jnp.maximum((hi - a + CHUNK_ROWS - 1) // CHUNK_ROWS, 1)

    def base_of(c):
        return pl.multiple_of(
            jnp.minimum(a + c * CHUNK_ROWS, NUM_ROWS - CHUNK_ROWS), 8
        )

    for k in range(NBUF - 1):

        @pl.when(k < nch)
        def _(k=k):
            pltpu.async_copy(
                x_hbm.at[pl.ds(base_of(k), CHUNK_ROWS)], buf.at[k], sems.at[k]
            )

    def chunk_body(c, carry):
        accs_a, accs_b = carry
        p = c % NBUF
        base = base_of(c)
        pltpu.make_async_copy(
            x_hbm.at[pl.ds(base, CHUNK_ROWS)], buf.at[p], sems.at[p]
        ).wait()

        @pl.when(c + NBUF - 1 < nch)
        def _():
            nxt = (c + NBUF - 1) % NBUF
            pltpu.async_copy(
                x_hbm.at[pl.ds(base_of(c + NBUF - 1), CHUNK_ROWS)],
                buf.at[nxt],
                sems.at[nxt],
            )

        i0 = jnp.maximum(lo - base, 0)
        i1 = jnp.minimum(hi - base, CHUNK_ROWS)
        im = jnp.minimum(jnp.maximum(mid - base, i0), i1)

        def row_a(i, accs):
            return tuple(
                jnp.maximum(accs[d], buf[p, i, pl.ds(LANES * d, LANES)])
                for d in range(VREGS_PER_ROW)
            )

        accs_a = plsc.parallel_loop(i0, im, 1, unroll=2, carry=accs_a)(row_a)
        accs_b = plsc.parallel_loop(im, i1, 1, unroll=2, carry=accs_b)(row_a)
        return (accs_a, accs_b)

    neg_inf = jnp.full((LANES,), -jnp.inf, dtype=jnp.float32)
    init = tuple(neg_inf for _ in range(VREGS_PER_ROW))
    accs_a, accs_b = lax.fori_loop(0, nch, chunk_body, (init, init))
    for d in range(VREGS_PER_ROW):
        out_v[0, pl.ds(LANES * d, LANES)] = accs_a[d]
        out_v[1, pl.ds(LANES * d, LANES)] = accs_b[d]

    pltpu.sync_copy(out_v, out_hbm.at[pl.ds(SEGS_PER_WORKER * w, SEGS_PER_WORKER)])


@jax.jit
def kernel(x, batch):
    batch = batch.astype(jnp.int32)
    queries = jnp.arange(NUM_SEGS + 1, dtype=jnp.int32)
    starts = jnp.searchsorted(
        batch, queries, side="left", method="compare_all"
    ).astype(jnp.int32)
    starts = jnp.full((80,), NUM_ROWS, dtype=jnp.int32).at[: NUM_SEGS + 1].set(starts)

    mesh = plsc.VectorSubcoreMesh(core_axis_name="c", subcore_axis_name="s")
    run = functools.partial(
        pl.kernel,
        mesh=mesh,
        out_type=jax.ShapeDtypeStruct((NUM_SEGS, NUM_COLS), jnp.float32),
        scratch_types=[
            pltpu.VMEM((80,), jnp.int32),
            pltpu.VMEM((NBUF, CHUNK_ROWS, NUM_COLS), jnp.float32),
            pltpu.VMEM((SEGS_PER_WORKER, NUM_COLS), jnp.float32),
            pltpu.SemaphoreType.DMA((NBUF,)),
        ],
    )(_sc_body)
    return run(x, starts)


# speedup vs baseline: 1.0126x; 1.0126x over previous
import functools

import jax
import jax.numpy as jnp
from jax import lax
from jax.experimental import pallas as pl
from jax.experimental.pallas import tpu as pltpu
from jax.experimental.pallas import tpu_sc as plsc

NUM_ROWS = 50000
NUM_COLS = 256
NUM_SEGS = 64
LANES = 16
VREGS_PER_ROW = NUM_COLS // LANES
NUM_CORES = 2
NUM_SUBCORES = 16
NUM_WORKERS = NUM_CORES * NUM_SUBCORES
SEGS_PER_WORKER = NUM_SEGS // NUM_WORKERS
CHUNK_ROWS = 48
NBUF = 5


def _sc_body(x_hbm, starts_hbm, out_hbm, starts_v, buf, out_v, sems):
    w = lax.axis_index("s") * NUM_CORES + lax.axis_index("c")
    pltpu.sync_copy(starts_hbm, starts_v)
    bounds = starts_v[pl.ds(SEGS_PER_WORKER * w, LANES)]

    lo = bounds[0]
    mid = bounds[1]
    hi = bounds[2]
    a = (lo // 8) * 8
    nch = jnp.maximum((hi - a + CHUNK_ROWS - 1) // CHUNK_ROWS, 1)

    def base_of(c):
        return pl.multiple_of(
            jnp.minimum(a + c * CHUNK_ROWS, NUM_ROWS - CHUNK_ROWS), 8
        )

    for k in range(NBUF - 1):

        @pl.when(k < nch)
        def _(k=k):
            pltpu.async_copy(
                x_hbm.at[pl.ds(base_of(k), CHUNK_ROWS)], buf.at[k], sems.at[k]
            )

    def chunk_body(c, carry):
        accs_a, accs_b = carry
        p = c % NBUF
        base = base_of(c)
        pltpu.make_async_copy(
            x_hbm.at[pl.ds(base, CHUNK_ROWS)], buf.at[p], sems.at[p]
        ).wait()

        @pl.when(c + NBUF - 1 < nch)
        def _():
            nxt = (c + NBUF - 1) % NBUF
            pltpu.async_copy(
                x_hbm.at[pl.ds(base_of(c + NBUF - 1), CHUNK_ROWS)],
                buf.at[nxt],
                sems.at[nxt],
            )

        i0 = jnp.maximum(lo - base, 0)
        i1 = jnp.minimum(hi - base, CHUNK_ROWS)
        im = jnp.minimum(jnp.maximum(mid - base, i0), i1)

        def row_a(i, accs):
            return tuple(
                jnp.maximum(accs[d], buf[p, i, pl.ds(LANES * d, LANES)])
                for d in range(VREGS_PER_ROW)
            )

        accs_a = plsc.parallel_loop(i0, im, 1, unroll=2, carry=accs_a)(row_a)
        accs_b = plsc.parallel_loop(im, i1, 1, unroll=2, carry=accs_b)(row_a)
        return (accs_a, accs_b)

    neg_inf = jnp.full((LANES,), -jnp.inf, dtype=jnp.float32)
    init = tuple(neg_inf for _ in range(VREGS_PER_ROW))
    accs_a, accs_b = lax.fori_loop(0, nch, chunk_body, (init, init))
    for d in range(VREGS_PER_ROW):
        out_v[0, pl.ds(LANES * d, LANES)] = accs_a[d]
        out_v[1, pl.ds(LANES * d, LANES)] = accs_b[d]

    pltpu.sync_copy(out_v, out_hbm.at[pl.ds(SEGS_PER_WORKER * w, SEGS_PER_WORKER)])


@jax.jit
def kernel(x, batch):
    batch = batch.astype(jnp.int32)
    queries = jnp.arange(NUM_SEGS + 1, dtype=jnp.int32)
    starts = jnp.searchsorted(
        batch, queries, side="left", method="compare_all"
    ).astype(jnp.int32)
    starts = jnp.full((80,), NUM_ROWS, dtype=jnp.int32).at[: NUM_SEGS + 1].set(starts)

    mesh = plsc.VectorSubcoreMesh(core_axis_name="c", subcore_axis_name="s")
    run = functools.partial(
        pl.kernel,
        mesh=mesh,
        out_type=jax.ShapeDtypeStruct((NUM_SEGS, NUM_COLS), jnp.float32),
        scratch_types=[
            pltpu.VMEM((80,), jnp.int32),
            pltpu.VMEM((NBUF, CHUNK_ROWS, NUM_COLS), jnp.float32),
            pltpu.VMEM((SEGS_PER_WORKER, NUM_COLS), jnp.float32),
            pltpu.SemaphoreType.DMA((NBUF,)),
        ],
    )(_sc_body)
    return run(x, starts)
